# baseline (speedup 1.0000x reference)
import functools

import jax
import jax.numpy as jnp
from jax import lax
from jax.experimental import pallas as pl
from jax.experimental.pallas import tpu as pltpu
from jax.experimental.pallas import tpu_sc as plsc

ROWS = 128
T_IN = 122880
T_OUT = 23552

NUM_CORES = 2
NUM_SUBCORES = 16
NUM_WORKERS = NUM_CORES * NUM_SUBCORES
ROWS_PER_WORKER = ROWS // NUM_WORKERS

SEG0 = (0, 8, 0, 4, 20480, 2560)
SEG1 = (81920, 4, 10240, 2, 18432, 4608)
COPY_SEG = (118784, 19456, 4096)

IN_BUF = 20480
OUT_BUF = 4608


def _decimator_body(x_hbm, out_hbm, in_v0, in_v1, out_v0, out_v1, tl_v0, tl_v1,
                    si0, si1, so0, so1, st0, st1):
  cid = lax.axis_index("c")
  sid = lax.axis_index("s")
  wid = cid * NUM_SUBCORES + sid
  row0 = wid * ROWS_PER_WORKER

  lanes = lax.iota(jnp.int32, 16)
  in_v = (in_v0, in_v1)
  out_v = (out_v0, out_v1)
  tl_v = (tl_v0, tl_v1)
  sin = (si0, si1)
  sout = (so0, so1)
  stl = (st0, st1)

  def run_segment(seg):
    in_off, stride, out_off, cpr, in_chunk, out_chunk = seg
    n = cpr * ROWS_PER_WORKER
    half = n // 2
    idx0 = lanes * stride
    step = 16 * stride
    n_gather = out_chunk // 16

    def chunk_row_off(i):
      r = row0 + i // cpr
      c = i % cpr
      return r, c

    def in_copy(i, b):
      r, c = chunk_row_off(i)
      return pltpu.make_async_copy(
          x_hbm.at[r, pl.ds(in_off + c * in_chunk, in_chunk)],
          in_v[b].at[pl.ds(0, in_chunk)],
          sin[b],
      )

    def out_copy(i, b):
      r, c = chunk_row_off(i)
      return pltpu.make_async_copy(
          out_v[b].at[pl.ds(0, out_chunk)],
          out_hbm.at[r, pl.ds(out_off + c * out_chunk, out_chunk)],
          sout[b],
      )

    def gather(b):
      src = in_v[b]
      dst = out_v[b]

      @plsc.parallel_loop(0, n_gather, unroll=8)
      def _(j):
        idx = idx0 + j * step
        vals = plsc.load_gather(src, [idx])
        dst[pl.ds(j * 16, 16)] = vals

    in_copy(0, 0).start()

    def loop_body(t, carry):
      i = 2 * t
      in_copy(i + 1, 1).start()

      @pl.when(t > 0)
      def _():
        out_copy(i - 2, 0).wait()
      in_copy(i, 0).wait()
      gather(0)
      out_copy(i, 0).start()

      @pl.when(t + 1 < half)
      def _():
        in_copy(i + 2, 0).start()

      @pl.when(t > 0)
      def _():
        out_copy(i - 1, 1).wait()
      in_copy(i + 1, 1).wait()
      gather(1)
      out_copy(i + 1, 1).start()
      return carry

    lax.fori_loop(0, half, loop_body, 0)
    out_copy(n - 2, 0).wait()
    out_copy(n - 1, 1).wait()

  in_off, out_off, length = COPY_SEG

  def tail_in(k):
    return pltpu.make_async_copy(
        x_hbm.at[row0 + k, pl.ds(in_off, length)],
        tl_v[k % 2].at[pl.ds(0, length)],
        stl[k % 2],
    )

  def tail_out(k):
    return pltpu.make_async_copy(
        tl_v[k % 2].at[pl.ds(0, length)],
        out_hbm.at[row0 + k, pl.ds(out_off, length)],
        sout[k % 2],
    )

  tail_in(0).start()
  tail_in(1).start()

  run_segment(SEG0)
  run_segment(SEG1)

  tail_in(0).wait()
  tail_out(0).start()
  tail_in(1).wait()
  tail_out(1).start()
  tail_out(0).wait()
  tail_in(2).start()
  tail_out(1).wait()
  tail_in(3).start()
  tail_in(2).wait()
  tail_out(2).start()
  tail_in(3).wait()
  tail_out(3).start()
  tail_out(2).wait()
  tail_out(3).wait()


@jax.jit
def _decimate(x2d):
  mesh = plsc.VectorSubcoreMesh(core_axis_name="c", subcore_axis_name="s")
  f = functools.partial(
      pl.kernel,
      mesh=mesh,
      out_type=jax.ShapeDtypeStruct((ROWS, T_OUT), jnp.float32),
      scratch_types=[
          pltpu.VMEM((IN_BUF,), jnp.float32),
          pltpu.VMEM((IN_BUF,), jnp.float32),
          pltpu.VMEM((OUT_BUF,), jnp.float32),
          pltpu.VMEM((OUT_BUF,), jnp.float32),
          pltpu.VMEM((4096,), jnp.float32),
          pltpu.VMEM((4096,), jnp.float32),
          pltpu.SemaphoreType.DMA,
          pltpu.SemaphoreType.DMA,
          pltpu.SemaphoreType.DMA,
          pltpu.SemaphoreType.DMA,
          pltpu.SemaphoreType.DMA,
          pltpu.SemaphoreType.DMA,
      ],
      compiler_params=pltpu.CompilerParams(needs_layout_passes=False),
  )(_decimator_body)
  return f(x2d)


def kernel(X):
  assert X.shape == (16, 8, T_IN), X.shape
  x2d = X.reshape(ROWS, T_IN)
  out = _decimate(x2d)
  return out.reshape(16, 8, T_OUT)


# speedup vs baseline: 1.0143x; 1.0143x over previous
import functools

import jax
import jax.numpy as jnp
from jax import lax
from jax.experimental import pallas as pl
from jax.experimental.pallas import tpu as pltpu
from jax.experimental.pallas import tpu_sc as plsc

ROWS = 128
T_IN = 122880
T_OUT = 23552

NUM_CORES = 2
NUM_SUBCORES = 16
NUM_WORKERS = NUM_CORES * NUM_SUBCORES
ROWS_PER_WORKER = ROWS // NUM_WORKERS

SEG0 = (0, 8, 0, 2, 40960, 5120)
SEG1 = (81920, 4, 10240, 1, 36864, 9216)
COPY_SEG = (118784, 19456, 4096)

IN_BUF = 40960
OUT_BUF = 9216


def _decimator_body(x_hbm, out_hbm, in_v0, in_v1, out_v0, out_v1, tl_v0, tl_v1,
                    si0, si1, so0, so1, st0, st1):
  cid = lax.axis_index("c")
  sid = lax.axis_index("s")
  wid = cid * NUM_SUBCORES + sid
  row0 = wid * ROWS_PER_WORKER

  lanes = lax.iota(jnp.int32, 16)
  in_v = (in_v0, in_v1)
  out_v = (out_v0, out_v1)
  tl_v = (tl_v0, tl_v1)
  sin = (si0, si1)
  sout = (so0, so1)
  stl = (st0, st1)

  def run_segment(seg):
    in_off, stride, out_off, cpr, in_chunk, out_chunk = seg
    n = cpr * ROWS_PER_WORKER
    half = n // 2
    idx0 = lanes * stride
    step = 16 * stride
    n_gather = out_chunk // 16

    def chunk_row_off(i):
      r = row0 + i // cpr
      c = i % cpr
      return r, c

    def in_copy(i, b):
      r, c = chunk_row_off(i)
      return pltpu.make_async_copy(
          x_hbm.at[r, pl.ds(in_off + c * in_chunk, in_chunk)],
          in_v[b].at[pl.ds(0, in_chunk)],
          sin[b],
      )

    def out_copy(i, b):
      r, c = chunk_row_off(i)
      return pltpu.make_async_copy(
          out_v[b].at[pl.ds(0, out_chunk)],
          out_hbm.at[r, pl.ds(out_off + c * out_chunk, out_chunk)],
          sout[b],
      )

    def gather(b):
      src = in_v[b]
      dst = out_v[b]

      @plsc.parallel_loop(0, n_gather, unroll=8)
      def _(j):
        idx = idx0 + j * step
        vals = plsc.load_gather(src, [idx])
        dst[pl.ds(j * 16, 16)] = vals

    in_copy(0, 0).start()

    def loop_body(t, carry):
      i = 2 * t
      in_copy(i + 1, 1).start()

      @pl.when(t > 0)
      def _():
        out_copy(i - 2, 0).wait()
      in_copy(i, 0).wait()
      gather(0)
      out_copy(i, 0).start()

      @pl.when(t + 1 < half)
      def _():
        in_copy(i + 2, 0).start()

      @pl.when(t > 0)
      def _():
        out_copy(i - 1, 1).wait()
      in_copy(i + 1, 1).wait()
      gather(1)
      out_copy(i + 1, 1).start()
      return carry

    lax.fori_loop(0, half, loop_body, 0)
    out_copy(n - 2, 0).wait()
    out_copy(n - 1, 1).wait()

  in_off, out_off, length = COPY_SEG

  def tail_in(k):
    return pltpu.make_async_copy(
        x_hbm.at[row0 + k, pl.ds(in_off, length)],
        tl_v[k % 2].at[pl.ds(0, length)],
        stl[k % 2],
    )

  def tail_out(k):
    return pltpu.make_async_copy(
        tl_v[k % 2].at[pl.ds(0, length)],
        out_hbm.at[row0 + k, pl.ds(out_off, length)],
        sout[k % 2],
    )

  tail_in(0).start()
  tail_in(1).start()

  run_segment(SEG0)
  run_segment(SEG1)

  tail_in(0).wait()
  tail_out(0).start()
  tail_in(1).wait()
  tail_out(1).start()
  tail_out(0).wait()
  tail_in(2).start()
  tail_out(1).wait()
  tail_in(3).start()
  tail_in(2).wait()
  tail_out(2).start()
  tail_in(3).wait()
  tail_out(3).start()
  tail_out(2).wait()
  tail_out(3).wait()


@jax.jit
def _decimate(x2d):
  mesh = plsc.VectorSubcoreMesh(core_axis_name="c", subcore_axis_name="s")
  f = functools.partial(
      pl.kernel,
      mesh=mesh,
      out_type=jax.ShapeDtypeStruct((ROWS, T_OUT), jnp.float32),
      scratch_types=[
          pltpu.VMEM((IN_BUF,), jnp.float32),
          pltpu.VMEM((IN_BUF,), jnp.float32),
          pltpu.VMEM((OUT_BUF,), jnp.float32),
          pltpu.VMEM((OUT_BUF,), jnp.float32),
          pltpu.VMEM((4096,), jnp.float32),
          pltpu.VMEM((4096,), jnp.float32),
          pltpu.SemaphoreType.DMA,
          pltpu.SemaphoreType.DMA,
          pltpu.SemaphoreType.DMA,
          pltpu.SemaphoreType.DMA,
          pltpu.SemaphoreType.DMA,
          pltpu.SemaphoreType.DMA,
      ],
      compiler_params=pltpu.CompilerParams(needs_layout_passes=False),
  )(_decimator_body)
  return f(x2d)


def kernel(X):
  assert X.shape == (16, 8, T_IN), X.shape
  x2d = X.reshape(ROWS, T_IN)
  out = _decimate(x2d)
  return out.reshape(16, 8, T_OUT)


# speedup vs baseline: 1.0626x; 1.0476x over previous
import functools

import jax
import jax.numpy as jnp
from jax import lax
from jax.experimental import pallas as pl
from jax.experimental.pallas import tpu as pltpu
from jax.experimental.pallas import tpu_sc as plsc

ROWS = 128
T_IN = 122880
T_OUT = 23552

NUM_CORES = 2
NUM_SUBCORES = 16
NUM_WORKERS = NUM_CORES * NUM_SUBCORES
ROWS_PER_WORKER = ROWS // NUM_WORKERS

SEG0 = (0, 8, 0, 4, 20480, 2560)
SEG1 = (81920, 4, 10240, 2, 18432, 4608)
COPY_SEG = (118784, 19456, 4096)

IN_BUF = 20480
OUT_BUF = 4608
D_IN = 4
D_OUT = 2


def _decimator_body(x_hbm, out_hbm,
                    iv0, iv1, iv2, iv3, ov0, ov1, tv0, tv1,
                    si0, si1, si2, si3, so0, so1, st0, st1):
  cid = lax.axis_index("c")
  sid = lax.axis_index("s")
  wid = cid * NUM_SUBCORES + sid
  row0 = wid * ROWS_PER_WORKER

  lanes = lax.iota(jnp.int32, 16)
  in_v = (iv0, iv1, iv2, iv3)
  out_v = (ov0, ov1)
  tl_v = (tv0, tv1)
  sin = (si0, si1, si2, si3)
  sout = (so0, so1)
  stl = (st0, st1)

  def run_segment(seg):
    in_off, stride, out_off, cpr, in_chunk, out_chunk = seg
    n = cpr * ROWS_PER_WORKER
    iters = n // D_IN
    idx0 = lanes * stride
    step = 16 * stride
    n_gather = out_chunk // 16

    def chunk_row_off(i):
      r = row0 + i // cpr
      c = i % cpr
      return r, c

    def in_copy(i, b):
      r, c = chunk_row_off(i)
      return pltpu.make_async_copy(
          x_hbm.at[r, pl.ds(in_off + c * in_chunk, in_chunk)],
          in_v[b].at[pl.ds(0, in_chunk)],
          sin[b],
      )

    def out_copy(i, b):
      r, c = chunk_row_off(i)
      return pltpu.make_async_copy(
          out_v[b].at[pl.ds(0, out_chunk)],
          out_hbm.at[r, pl.ds(out_off + c * out_chunk, out_chunk)],
          sout[b],
      )

    def gather(bi, bo):
      src = in_v[bi]
      dst = out_v[bo]

      @plsc.parallel_loop(0, n_gather, unroll=8)
      def _(j):
        idx = idx0 + j * step
        vals = plsc.load_gather(src, [idx])
        dst[pl.ds(j * 16, 16)] = vals

    for j in range(D_IN):
      in_copy(j, j).start()

    def loop_body(t, carry):
      for j in range(D_IN):
        i = D_IN * t + j
        bo = j % D_OUT

        @pl.when(jnp.logical_or(t > 0, j >= D_OUT))
        def _():
          out_copy(i - D_OUT, bo).wait()
        in_copy(i, j).wait()
        gather(j, bo)
        out_copy(i, bo).start()

        @pl.when(t + 1 < iters)
        def _():
          in_copy(i + D_IN, j).start()
      return carry

    lax.fori_loop(0, iters, loop_body, 0)
    out_copy(n - 2, (n - 2) % D_OUT).wait()
    out_copy(n - 1, (n - 1) % D_OUT).wait()

  in_off, out_off, length = COPY_SEG

  def tail_in(k):
    return pltpu.make_async_copy(
        x_hbm.at[row0 + k, pl.ds(in_off, length)],
        tl_v[k % 2].at[pl.ds(0, length)],
        stl[k % 2],
    )

  def tail_out(k):
    return pltpu.make_async_copy(
        tl_v[k % 2].at[pl.ds(0, length)],
        out_hbm.at[row0 + k, pl.ds(out_off, length)],
        sout[k % 2],
    )

  tail_in(0).start()
  tail_in(1).start()

  run_segment(SEG0)
  run_segment(SEG1)

  tail_in(0).wait()
  tail_out(0).start()
  tail_in(1).wait()
  tail_out(1).start()
  tail_out(0).wait()
  tail_in(2).start()
  tail_out(1).wait()
  tail_in(3).start()
  tail_in(2).wait()
  tail_out(2).start()
  tail_in(3).wait()
  tail_out(3).start()
  tail_out(2).wait()
  tail_out(3).wait()


@jax.jit
def _decimate(x2d):
  mesh = plsc.VectorSubcoreMesh(core_axis_name="c", subcore_axis_name="s")
  f = functools.partial(
      pl.kernel,
      mesh=mesh,
      out_type=jax.ShapeDtypeStruct((ROWS, T_OUT), jnp.float32),
      scratch_types=[
          pltpu.VMEM((IN_BUF,), jnp.float32),
          pltpu.VMEM((IN_BUF,), jnp.float32),
          pltpu.VMEM((IN_BUF,), jnp.float32),
          pltpu.VMEM((IN_BUF,), jnp.float32),
          pltpu.VMEM((OUT_BUF,), jnp.float32),
          pltpu.VMEM((OUT_BUF,), jnp.float32),
          pltpu.VMEM((4096,), jnp.float32),
          pltpu.VMEM((4096,), jnp.float32),
          pltpu.SemaphoreType.DMA,
          pltpu.SemaphoreType.DMA,
          pltpu.SemaphoreType.DMA,
          pltpu.SemaphoreType.DMA,
          pltpu.SemaphoreType.DMA,
          pltpu.SemaphoreType.DMA,
          pltpu.SemaphoreType.DMA,
          pltpu.SemaphoreType.DMA,
      ],
      compiler_params=pltpu.CompilerParams(needs_layout_passes=False),
  )(_decimator_body)
  return f(x2d)


def kernel(X):
  assert X.shape == (16, 8, T_IN), X.shape
  x2d = X.reshape(ROWS, T_IN)
  out = _decimate(x2d)
  return out.reshape(16, 8, T_OUT)


# speedup vs baseline: 1.0737x; 1.0105x over previous
import functools

import jax
import jax.numpy as jnp
from jax import lax
from jax.experimental import pallas as pl
from jax.experimental.pallas import tpu as pltpu
from jax.experimental.pallas import tpu_sc as plsc

ROWS = 128
T_IN = 122880
T_OUT = 23552

NUM_CORES = 2
NUM_SUBCORES = 16
NUM_WORKERS = NUM_CORES * NUM_SUBCORES
ROWS_PER_WORKER = ROWS // NUM_WORKERS

SEG0 = (0, 8, 0, 5, 16384, 2048, 5)
SEG1 = (81920, 4, 10240, 2, 18432, 4608, 4)
COPY_SEG = (118784, 19456, 4096)

IN_BUF = 18432
OUT_BUF = 4608
D_OUT = 2


def _decimator_body(x_hbm, out_hbm,
                    iv0, iv1, iv2, iv3, iv4, ov0, ov1, tv0, tv1,
                    si0, si1, si2, si3, si4, so0, so1, st0, st1):
  cid = lax.axis_index("c")
  sid = lax.axis_index("s")
  wid = cid * NUM_SUBCORES + sid
  row0 = wid * ROWS_PER_WORKER

  lanes = lax.iota(jnp.int32, 16)
  in_v = (iv0, iv1, iv2, iv3, iv4)
  out_v = (ov0, ov1)
  tl_v = (tv0, tv1)
  sin = (si0, si1, si2, si3, si4)
  sout = (so0, so1)
  stl = (st0, st1)

  def run_segment(seg):
    in_off, stride, out_off, cpr, in_chunk, out_chunk, d_in = seg
    n = cpr * ROWS_PER_WORKER
    iters = n // d_in
    idx0 = lanes * stride
    step = 16 * stride
    n_gather = out_chunk // 16

    def chunk_row_off(i):
      r = row0 + i // cpr
      c = i % cpr
      return r, c

    def in_copy(i, b):
      r, c = chunk_row_off(i)
      return pltpu.make_async_copy(
          x_hbm.at[r, pl.ds(in_off + c * in_chunk, in_chunk)],
          in_v[b].at[pl.ds(0, in_chunk)],
          sin[b],
      )

    def out_copy(i, b):
      r, c = chunk_row_off(i)
      return pltpu.make_async_copy(
          out_v[b].at[pl.ds(0, out_chunk)],
          out_hbm.at[r, pl.ds(out_off + c * out_chunk, out_chunk)],
          sout[b],
      )

    def gather(bi, bo):
      src = in_v[bi]
      dst = out_v[bo]

      @plsc.parallel_loop(0, n_gather, unroll=8)
      def _(j):
        idx = idx0 + j * step
        vals = plsc.load_gather(src, [idx])
        dst[pl.ds(j * 16, 16)] = vals

    def prev_dist(j):
      b = j % D_OUT
      prevs = [jj for jj in range(j) if jj % D_OUT == b]
      if prevs:
        return j - prevs[-1], True
      lasts = [jj for jj in range(d_in) if jj % D_OUT == b]
      return j + d_in - lasts[-1], False

    for j in range(d_in):
      in_copy(j, j).start()

    def loop_body(t, carry):
      for j in range(d_in):
        i = d_in * t + j
        bo = j % D_OUT
        dist, same_iter = prev_dist(j)

        if same_iter:
          out_copy(i - dist, bo).wait()
        else:
          @pl.when(t > 0)
          def _():
            out_copy(i - dist, bo).wait()
        in_copy(i, j).wait()
        gather(j, bo)
        out_copy(i, bo).start()

        @pl.when(t + 1 < iters)
        def _():
          in_copy(i + d_in, j).start()
      return carry

    lax.fori_loop(0, iters, loop_body, 0)
    for b in range(D_OUT):
      j_b = max(jj for jj in range(d_in) if jj % D_OUT == b)
      out_copy(n - d_in + j_b, b).wait()

  in_off, out_off, length = COPY_SEG

  def tail_in(k):
    return pltpu.make_async_copy(
        x_hbm.at[row0 + k, pl.ds(in_off, length)],
        tl_v[k % 2].at[pl.ds(0, length)],
        stl[k % 2],
    )

  def tail_out(k):
    return pltpu.make_async_copy(
        tl_v[k % 2].at[pl.ds(0, length)],
        out_hbm.at[row0 + k, pl.ds(out_off, length)],
        sout[k % 2],
    )

  tail_in(0).start()
  tail_in(1).start()

  run_segment(SEG0)
  run_segment(SEG1)

  tail_in(0).wait()
  tail_out(0).start()
  tail_in(1).wait()
  tail_out(1).start()
  tail_out(0).wait()
  tail_in(2).start()
  tail_out(1).wait()
  tail_in(3).start()
  tail_in(2).wait()
  tail_out(2).start()
  tail_in(3).wait()
  tail_out(3).start()
  tail_out(2).wait()
  tail_out(3).wait()


@jax.jit
def _decimate(x2d):
  mesh = plsc.VectorSubcoreMesh(core_axis_name="c", subcore_axis_name="s")
  f = functools.partial(
      pl.kernel,
      mesh=mesh,
      out_type=jax.ShapeDtypeStruct((ROWS, T_OUT), jnp.float32),
      scratch_types=[
          pltpu.VMEM((IN_BUF,), jnp.float32),
          pltpu.VMEM((IN_BUF,), jnp.float32),
          pltpu.VMEM((IN_BUF,), jnp.float32),
          pltpu.VMEM((IN_BUF,), jnp.float32),
          pltpu.VMEM((IN_BUF,), jnp.float32),
          pltpu.VMEM((OUT_BUF,), jnp.float32),
          pltpu.VMEM((OUT_BUF,), jnp.float32),
          pltpu.VMEM((4096,), jnp.float32),
          pltpu.VMEM((4096,), jnp.float32),
          pltpu.SemaphoreType.DMA,
          pltpu.SemaphoreType.DMA,
          pltpu.SemaphoreType.DMA,
          pltpu.SemaphoreType.DMA,
          pltpu.SemaphoreType.DMA,
          pltpu.SemaphoreType.DMA,
          pltpu.SemaphoreType.DMA,
          pltpu.SemaphoreType.DMA,
          pltpu.SemaphoreType.DMA,
      ],
      compiler_params=pltpu.CompilerParams(needs_layout_passes=False),
  )(_decimator_body)
  return f(x2d)


def kernel(X):
  assert X.shape == (16, 8, T_IN), X.shape
  x2d = X.reshape(ROWS, T_IN)
  out = _decimate(x2d)
  return out.reshape(16, 8, T_OUT)
